# baseline (speedup 1.0000x reference)
import functools

import jax
import jax.numpy as jnp
from jax import lax
from jax.experimental import pallas as pl
from jax.experimental.pallas import tpu as pltpu
from jax.experimental.pallas import tpu_sc as plsc

VOCAB = 49408
EMBED = 512
NUM_POS = 77
BATCH = 1024
SEQ = 77
TOTAL = BATCH * SEQ
NUM_WORKERS = 32
ROWS_PER_WORKER = TOTAL // NUM_WORKERS
CHUNK = 56
CHUNKS_PER_WORKER = ROWS_PER_WORKER // CHUNK
POS_EXT = NUM_POS + CHUNK - 1
LANES = 16
VREGS_PER_ROW = EMBED // LANES


def _emb_body(ids_hbm, table_hbm, posx_hbm, out_hbm, idx_v, rows_v, pos_v, gsem):
  num_cores = 2
  wid = lax.axis_index("s") * num_cores + lax.axis_index("c")
  base = wid * ROWS_PER_WORKER

  pltpu.sync_copy(ids_hbm.at[pl.ds(base, ROWS_PER_WORKER)], idx_v)
  pltpu.sync_copy(posx_hbm, pos_v)

  def chunk_body(c, carry):
    r0 = c * CHUNK
    pltpu.async_copy(
        table_hbm.at[idx_v.at[pl.ds(r0, CHUNK)]], rows_v, gsem
    ).wait()

    s0 = lax.rem(base + r0, NUM_POS)

    def row_body(k, cc):
      for j in range(VREGS_PER_ROW):
        sl = pl.ds(j * LANES, LANES)
        rows_v[k, sl] = rows_v[k, sl] + pos_v[s0 + k, sl]
      return cc

    lax.fori_loop(0, CHUNK, row_body, 0, unroll=False)

    pltpu.sync_copy(rows_v, out_hbm.at[pl.ds(base + r0, CHUNK)])
    return carry

  lax.fori_loop(0, CHUNKS_PER_WORKER, chunk_body, 0, unroll=False)


@jax.jit
def kernel(input_ids, token_embedding, position_embedding):
  ids_flat = input_ids.astype(jnp.int32).reshape(TOTAL)
  pos_ext = jnp.concatenate(
      [position_embedding, position_embedding[: POS_EXT - NUM_POS]], axis=0
  )

  mesh = plsc.VectorSubcoreMesh(core_axis_name="c", subcore_axis_name="s")
  f = pl.kernel(
      _emb_body,
      out_type=jax.ShapeDtypeStruct((TOTAL, EMBED), jnp.float32),
      mesh=mesh,
      scratch_types=[
          pltpu.VMEM((ROWS_PER_WORKER,), jnp.int32),
          pltpu.VMEM((CHUNK, EMBED), jnp.float32),
          pltpu.VMEM((POS_EXT, EMBED), jnp.float32),
          pltpu.SemaphoreType.DMA,
      ],
  )
  out = f(ids_flat, token_embedding, pos_ext)
  return out.reshape(BATCH, SEQ, EMBED)


# speedup vs baseline: 1.2035x; 1.2035x over previous
import functools

import jax
import jax.numpy as jnp
from jax import lax
from jax.experimental import pallas as pl
from jax.experimental.pallas import tpu as pltpu
from jax.experimental.pallas import tpu_sc as plsc

VOCAB = 49408
EMBED = 512
NUM_POS = 77
BATCH = 1024
SEQ = 77
TOTAL = BATCH * SEQ
NUM_WORKERS = 32
ROWS_PER_WORKER = TOTAL // NUM_WORKERS
CHUNK = 56
NCHUNKS = ROWS_PER_WORKER // CHUNK
NBUF = 3
LANES = 16
VREGS_PER_ROW = EMBED // LANES
POS_PAD = 80


def _emb_body(ids_hbm, table_hbm, pos_hbm, out_hbm, idx_v, b0, b1, b2, pos_v,
              g0, g1, g2, w0, w1, w2):
  bufs = (b0, b1, b2)
  gsems = (g0, g1, g2)
  wsems = (w0, w1, w2)
  num_cores = 2
  wid = lax.axis_index("s") * num_cores + lax.axis_index("c")
  base = wid * ROWS_PER_WORKER

  pltpu.sync_copy(ids_hbm.at[pl.ds(base, ROWS_PER_WORKER)], idx_v)
  pltpu.sync_copy(pos_hbm, pos_v)

  def gather_start(c, b):
    pltpu.async_copy(
        table_hbm.at[idx_v.at[pl.ds(c * CHUNK, CHUNK)]], bufs[b], gsems[b]
    )

  def gather_wait(b):
    pltpu.make_async_copy(
        table_hbm.at[idx_v.at[pl.ds(0, CHUNK)]], bufs[b], gsems[b]
    ).wait()

  def write_start(c, b):
    pltpu.async_copy(bufs[b], out_hbm.at[pl.ds(base + c * CHUNK, CHUNK)],
                     wsems[b])

  def write_wait(b):
    pltpu.make_async_copy(
        bufs[b], out_hbm.at[pl.ds(0, CHUNK)], wsems[b]
    ).wait()

  def add_pos(c, buf):
    s0 = lax.rem(c * CHUNK, NUM_POS)
    kw = jnp.minimum(NUM_POS - s0, CHUNK)

    def body1(k, cc):
      for j in range(VREGS_PER_ROW):
        sl = pl.ds(j * LANES, LANES)
        buf[k, sl] = buf[k, sl] + pos_v[s0 + k, sl]
      return cc

    lax.fori_loop(0, kw, body1, 0)

    def body2(k, cc):
      for j in range(VREGS_PER_ROW):
        sl = pl.ds(j * LANES, LANES)
        buf[k, sl] = buf[k, sl] + pos_v[s0 + k - NUM_POS, sl]
      return cc

    lax.fori_loop(kw, CHUNK, body2, 0)

  def step(c, b, first):
    gather_wait(b)
    add_pos(c, bufs[b])
    write_start(c, b)
    if not first:
      write_wait((b + 2) % NBUF)
    gather_start(c + 2, (b + 2) % NBUF)

  gather_start(0, 0)
  gather_start(1, 1)
  step(0, 0, True)
  step(1, 1, False)
  step(2, 2, False)

  def triple(i, carry):
    c0 = 3 * i
    for b in range(NBUF):
      step(c0 + b, b, False)
    return carry

  lax.fori_loop(1, NCHUNKS // NBUF, triple, 0)

  for c in (NCHUNKS - 2, NCHUNKS - 1):
    b = c % NBUF
    gather_wait(b)
    add_pos(c, bufs[b])
    write_start(c, b)
  for b in range(NBUF):
    write_wait(b)


@jax.jit
def kernel(input_ids, token_embedding, position_embedding):
  ids_flat = input_ids.astype(jnp.int32).reshape(TOTAL)

  mesh = plsc.VectorSubcoreMesh(core_axis_name="c", subcore_axis_name="s")
  f = pl.kernel(
      _emb_body,
      out_type=jax.ShapeDtypeStruct((TOTAL, EMBED), jnp.float32),
      mesh=mesh,
      scratch_types=[
          pltpu.VMEM((ROWS_PER_WORKER,), jnp.int32),
          pltpu.VMEM((CHUNK, EMBED), jnp.float32),
          pltpu.VMEM((CHUNK, EMBED), jnp.float32),
          pltpu.VMEM((CHUNK, EMBED), jnp.float32),
          pltpu.VMEM((NUM_POS, EMBED), jnp.float32),
          pltpu.SemaphoreType.DMA,
          pltpu.SemaphoreType.DMA,
          pltpu.SemaphoreType.DMA,
          pltpu.SemaphoreType.DMA,
          pltpu.SemaphoreType.DMA,
          pltpu.SemaphoreType.DMA,
      ],
  )
  out = f(ids_flat, token_embedding, position_embedding)
  return out.reshape(BATCH, SEQ, EMBED)
